# Initial kernel scaffold; baseline (speedup 1.0000x reference)
import jax
import jax.numpy as jnp
from jax.experimental import pallas as pl


def kernel(X_cat, X_num, tables, W1, b1, W2, b2):
    raise NotImplementedError("write your pallas kernel here")


# speedup vs baseline: 2.1310x; 2.1310x over previous
import functools

import jax
import jax.numpy as jnp
from jax import lax
from jax.experimental import pallas as pl
from jax.experimental.pallas import tpu as pltpu
from jax.experimental.pallas import tpu_sc as plsc

NUM_CORES = 2
NUM_SUBCORES = 16
NW = NUM_CORES * NUM_SUBCORES


def _make_sc_gather(total_rows: int, emb_dim: int, chunk: int):
    rows_per_w = total_rows // NW
    assert rows_per_w * NW == total_rows
    assert rows_per_w % chunk == 0
    nchunk = rows_per_w // chunk
    mesh = plsc.VectorSubcoreMesh(core_axis_name="c", subcore_axis_name="s")

    @functools.partial(
        pl.kernel,
        out_type=jax.ShapeDtypeStruct((total_rows, emb_dim), jnp.float32),
        mesh=mesh,
        scratch_types=[
            pltpu.VMEM((chunk,), jnp.int32),
            pltpu.VMEM((chunk, emb_dim), jnp.float32),
            pltpu.SemaphoreType.DMA,
        ],
        compiler_params=pltpu.CompilerParams(use_tc_tiling_on_sc=False),
    )
    def sc_gather(tables_hbm, idx_hbm, out_hbm, idx_v, rows_v, sem):
        wid = lax.axis_index("s") * NUM_CORES + lax.axis_index("c")
        base = wid * rows_per_w

        def chunk_body(k, carry):
            off = base + k * chunk
            pltpu.sync_copy(idx_hbm.at[pl.ds(off, chunk)], idx_v)
            pltpu.async_copy(tables_hbm.at[idx_v], rows_v, sem).wait()
            pltpu.sync_copy(rows_v, out_hbm.at[pl.ds(off, chunk)])
            return carry

        lax.fori_loop(0, nchunk, chunk_body, 0)

    return sc_gather


def _mlp_body(cat_ref, num_ref, w1c_ref, w1n_ref, b1_ref, w2_ref, b2_ref,
              out_ref):
    h = jnp.dot(cat_ref[...], w1c_ref[...], preferred_element_type=jnp.float32)
    h = h + jnp.dot(num_ref[...], w1n_ref[...],
                    preferred_element_type=jnp.float32)
    h = jnp.maximum(h + b1_ref[...], 0.0)
    out_ref[...] = (
        jnp.dot(h, w2_ref[...], preferred_element_type=jnp.float32)
        + b2_ref[...]
    )


def _tc_mlp(cat_emb, x_num, w1c, w1n, b1, w2, b2, blk: int):
    b_rows = cat_emb.shape[0]
    grid = (b_rows // blk,)
    return pl.pallas_call(
        _mlp_body,
        grid=grid,
        in_specs=[
            pl.BlockSpec((blk, cat_emb.shape[1]), lambda i: (i, 0)),
            pl.BlockSpec((blk, x_num.shape[1]), lambda i: (i, 0)),
            pl.BlockSpec(w1c.shape, lambda i: (0, 0)),
            pl.BlockSpec(w1n.shape, lambda i: (0, 0)),
            pl.BlockSpec(b1.shape, lambda i: (0, 0)),
            pl.BlockSpec(w2.shape, lambda i: (0, 0)),
            pl.BlockSpec(b2.shape, lambda i: (0, 0)),
        ],
        out_specs=pl.BlockSpec((blk, 1), lambda i: (i, 0)),
        out_shape=jax.ShapeDtypeStruct((b_rows, 1), jnp.float32),
    )(cat_emb, x_num, w1c, w1n, b1, w2, b2)


def kernel(X_cat, X_num, tables, W1, b1, W2, b2):
    b_rows, num_fields = X_cat.shape
    vocab, emb_dim = tables.shape[1], tables.shape[2]
    total_rows = b_rows * num_fields

    tables_flat = tables.reshape(num_fields * vocab, emb_dim)
    offsets = (jnp.arange(num_fields, dtype=jnp.int32) * vocab)[None, :]
    idx_flat = (X_cat.astype(jnp.int32) + offsets).reshape(total_rows)

    gathered = _make_sc_gather(total_rows, emb_dim, chunk=1664)(
        tables_flat, idx_flat)
    cat_emb = gathered.reshape(b_rows, num_fields * emb_dim)

    w1c = W1[:, : num_fields * emb_dim].T
    w1n = W1[:, num_fields * emb_dim:].T
    out = _tc_mlp(cat_emb, X_num, w1c, w1n, b1[None, :], W2.T,
                  b2[None, :], blk=2048)
    return out[:, 0]
